# baseline (device time: 16169 ns/iter reference)
import jax
import jax.numpy as jnp
from jax import lax
from jax.experimental import pallas as pl
from jax.experimental.pallas import tpu as pltpu

N_DEV = 4
N_EXP = 8
E_PER = N_EXP // N_DEV


def kernel(x, router_W, route_idx, expert_W, shared_W):
    n_tok, d = x.shape
    e_per, _, h_dim = expert_W.shape

    def body(x_ref, router_ref, idx_ref, ew_ref, sw_ref, out_ref,
             comm_ref, send_sems, recv_sems):
        my_pos = lax.axis_index("i")
        left = lax.rem(my_pos + N_DEV - 1, N_DEV)
        right = lax.rem(my_pos + 1, N_DEV)

        barrier_sem = pltpu.get_barrier_semaphore()
        for nbr in [left, right]:
            pl.semaphore_signal(
                barrier_sem, inc=1,
                device_id=(nbr,), device_id_type=pl.DeviceIdType.MESH,
            )
        pl.semaphore_wait(barrier_sem, 2)

        comm_ref[0] = ew_ref[...].astype(jnp.bfloat16)

        def hop_rdma(h):
            return pltpu.make_async_remote_copy(
                src_ref=comm_ref.at[h],
                dst_ref=comm_ref.at[h + 1],
                send_sem=send_sems.at[h],
                recv_sem=recv_sems.at[h],
                device_id=(right,),
                device_id_type=pl.DeviceIdType.MESH,
            )

        rdma0 = hop_rdma(0)
        rdma0.start()

        x_f32 = x_ref[...]
        x_bf = x_f32.astype(jnp.bfloat16)
        idx = idx_ref[...]

        scores = jnp.dot(x_f32, router_ref[...],
                         preferred_element_type=jnp.float32)
        s_max = jnp.max(scores, axis=-1, keepdims=True)
        p = jnp.exp(scores - s_max)
        probs = p / jnp.sum(p, axis=-1, keepdims=True)
        p_tok = jnp.zeros((n_tok, 1), dtype=jnp.float32)
        for k in range(N_EXP):
            p_tok = p_tok + jnp.where(idx == k, probs[:, k:k + 1], 0.0)

        acc = jnp.dot(x_bf, sw_ref[...].astype(jnp.bfloat16),
                      preferred_element_type=jnp.float32)

        def expert_contrib(acc, slot, origin):
            for j in range(E_PER):
                e = origin * E_PER + j
                y = jnp.dot(x_bf, comm_ref[slot, j],
                            preferred_element_type=jnp.float32)
                gate = jnp.where(idx == e, p_tok, 0.0)
                acc = acc + gate * y
            return acc

        rdma = rdma0
        for h in range(N_DEV - 1):
            origin = lax.rem(my_pos - h + N_DEV, N_DEV)
            acc = expert_contrib(acc, h, origin)
            rdma.wait()
            if h + 1 < N_DEV - 1:
                rdma = hop_rdma(h + 1)
                rdma.start()
        origin = lax.rem(my_pos - (N_DEV - 1) + N_DEV, N_DEV)
        acc = expert_contrib(acc, N_DEV - 1, origin)

        out_ref[...] = acc

    return pl.pallas_call(
        body,
        out_shape=jax.ShapeDtypeStruct((n_tok, h_dim), jnp.float32),
        in_specs=[pl.BlockSpec(memory_space=pltpu.VMEM)] * 5,
        out_specs=pl.BlockSpec(memory_space=pltpu.VMEM),
        scratch_shapes=[
            pltpu.VMEM((N_DEV, e_per, d, h_dim), jnp.bfloat16),
            pltpu.SemaphoreType.DMA((N_DEV - 1,)),
            pltpu.SemaphoreType.DMA((N_DEV - 1,)),
        ],
        compiler_params=pltpu.CompilerParams(collective_id=0),
    )(x, router_W, route_idx, expert_W, shared_W)


# device time: 11562 ns/iter; 1.3985x vs baseline; 1.3985x over previous
import jax
import jax.numpy as jnp
from jax import lax
from jax.experimental import pallas as pl
from jax.experimental.pallas import tpu as pltpu

N_DEV = 4
N_EXP = 8
E_PER = N_EXP // N_DEV


def kernel(x, router_W, route_idx, expert_W, shared_W):
    n_tok, d = x.shape
    e_per, _, h_dim = expert_W.shape

    def body(x_ref, router_ref, idx_ref, ew_ref, sw_ref, out_ref,
             comm_ref, send_sems, recv_sems):
        my_pos = lax.axis_index("i")

        barrier_sem = pltpu.get_barrier_semaphore()
        for o in range(1, N_DEV):
            peer = lax.rem(my_pos + o, N_DEV)
            pl.semaphore_signal(
                barrier_sem, inc=1,
                device_id=(peer,), device_id_type=pl.DeviceIdType.MESH,
            )
        pl.semaphore_wait(barrier_sem, N_DEV - 1)

        comm_ref[0] = ew_ref[...].astype(jnp.bfloat16)

        rdmas = []
        for o in range(1, N_DEV):
            tgt = lax.rem(my_pos + o, N_DEV)
            rdma = pltpu.make_async_remote_copy(
                src_ref=comm_ref.at[0],
                dst_ref=comm_ref.at[o],
                send_sem=send_sems.at[o - 1],
                recv_sem=recv_sems.at[o - 1],
                device_id=(tgt,),
                device_id_type=pl.DeviceIdType.MESH,
            )
            rdma.start()
            rdmas.append(rdma)

        x_f32 = x_ref[...]
        x_bf = x_f32.astype(jnp.bfloat16)
        idx = idx_ref[...]

        scores = jnp.dot(x_f32, router_ref[...],
                         preferred_element_type=jnp.float32)
        s_max = jnp.max(scores, axis=-1, keepdims=True)
        p = jnp.exp(scores - s_max)
        probs = p / jnp.sum(p, axis=-1, keepdims=True)
        p_tok = jnp.zeros((n_tok, 1), dtype=jnp.float32)
        for k in range(N_EXP):
            p_tok = p_tok + jnp.where(idx == k, probs[:, k:k + 1], 0.0)

        acc = jnp.dot(x_bf, sw_ref[...].astype(jnp.bfloat16),
                      preferred_element_type=jnp.float32)

        def expert_contrib(acc, slot, origin):
            for j in range(E_PER):
                e = origin * E_PER + j
                y = jnp.dot(x_bf, comm_ref[slot, j],
                            preferred_element_type=jnp.float32)
                gate = jnp.where(idx == e, p_tok, 0.0)
                acc = acc + gate * y
            return acc

        acc = expert_contrib(acc, 0, my_pos)
        for o in range(1, N_DEV):
            origin = lax.rem(my_pos - o + N_DEV, N_DEV)
            rdmas[o - 1].wait_recv()
            acc = expert_contrib(acc, o, origin)
        for rdma in rdmas:
            rdma.wait_send()

        out_ref[...] = acc

    return pl.pallas_call(
        body,
        out_shape=jax.ShapeDtypeStruct((n_tok, h_dim), jnp.float32),
        in_specs=[pl.BlockSpec(memory_space=pltpu.VMEM)] * 5,
        out_specs=pl.BlockSpec(memory_space=pltpu.VMEM),
        scratch_shapes=[
            pltpu.VMEM((N_DEV, e_per, d, h_dim), jnp.bfloat16),
            pltpu.SemaphoreType.DMA((N_DEV - 1,)),
            pltpu.SemaphoreType.DMA((N_DEV - 1,)),
        ],
        compiler_params=pltpu.CompilerParams(collective_id=0),
    )(x, router_W, route_idx, expert_W, shared_W)


# device time: 10446 ns/iter; 1.5479x vs baseline; 1.1068x over previous
import jax
import jax.numpy as jnp
from jax import lax
from jax.experimental import pallas as pl
from jax.experimental.pallas import tpu as pltpu

N_DEV = 4
N_EXP = 8
E_PER = N_EXP // N_DEV


def kernel(x, router_W, route_idx, expert_W, shared_W):
    n_tok, d = x.shape
    e_per, _, h_dim = expert_W.shape
    blk = e_per * d
    k_tot = d + N_DEV * blk

    def body(x_ref, router_ref, idx_ref, ew_ref, sw_ref, out_ref,
             w_ref, x_cat_ref, send_sems, recv_sems):
        my_pos = lax.axis_index("i")

        barrier_sem = pltpu.get_barrier_semaphore()
        for o in range(1, N_DEV):
            peer = lax.rem(my_pos + o, N_DEV)
            pl.semaphore_signal(
                barrier_sem, inc=1,
                device_id=(peer,), device_id_type=pl.DeviceIdType.MESH,
            )
        pl.semaphore_wait(barrier_sem, N_DEV - 1)

        w_ref[pl.ds(d, blk)] = (
            ew_ref[...].astype(jnp.bfloat16).reshape(blk, h_dim)
        )

        rdmas = []
        for o in range(1, N_DEV):
            tgt = lax.rem(my_pos + o, N_DEV)
            rdma = pltpu.make_async_remote_copy(
                src_ref=w_ref.at[pl.ds(d, blk)],
                dst_ref=w_ref.at[pl.ds(d + o * blk, blk)],
                send_sem=send_sems.at[o - 1],
                recv_sem=recv_sems.at[o - 1],
                device_id=(tgt,),
                device_id_type=pl.DeviceIdType.MESH,
            )
            rdma.start()
            rdmas.append(rdma)

        w_ref[pl.ds(0, d)] = sw_ref[...].astype(jnp.bfloat16)

        x_f32 = x_ref[...]
        idx = idx_ref[...]

        scores = jnp.dot(x_f32, router_ref[...],
                         preferred_element_type=jnp.float32)
        s_max = jnp.max(scores, axis=-1, keepdims=True)
        p = jnp.exp(scores - s_max)
        probs = p / jnp.sum(p, axis=-1, keepdims=True)
        p_tok = jnp.zeros((n_tok, 1), dtype=jnp.float32)
        for k in range(N_EXP):
            p_tok = p_tok + jnp.where(idx == k, probs[:, k:k + 1], 0.0)

        x_cat_ref[:, pl.ds(0, d)] = x_f32.astype(jnp.bfloat16)
        xg = (x_f32 * p_tok).astype(jnp.bfloat16)
        zero = jnp.zeros_like(xg)
        for o in range(N_DEV):
            origin = lax.rem(my_pos - o + N_DEV, N_DEV)
            for j in range(E_PER):
                e = origin * E_PER + j
                col = d + (o * E_PER + j) * d
                x_cat_ref[:, pl.ds(col, d)] = jnp.where(idx == e, xg, zero)

        for rdma in rdmas:
            rdma.wait_recv()
        out_ref[...] = jnp.dot(x_cat_ref[...], w_ref[...],
                               preferred_element_type=jnp.float32)
        for rdma in rdmas:
            rdma.wait_send()

    return pl.pallas_call(
        body,
        out_shape=jax.ShapeDtypeStruct((n_tok, h_dim), jnp.float32),
        in_specs=[pl.BlockSpec(memory_space=pltpu.VMEM)] * 5,
        out_specs=pl.BlockSpec(memory_space=pltpu.VMEM),
        scratch_shapes=[
            pltpu.VMEM((k_tot, h_dim), jnp.bfloat16),
            pltpu.VMEM((n_tok, k_tot), jnp.bfloat16),
            pltpu.SemaphoreType.DMA((N_DEV - 1,)),
            pltpu.SemaphoreType.DMA((N_DEV - 1,)),
        ],
        compiler_params=pltpu.CompilerParams(collective_id=0),
    )(x, router_W, route_idx, expert_W, shared_W)
